# baseline (speedup 1.0000x reference)
import functools

import jax
import jax.numpy as jnp
from jax import lax
from jax.experimental import pallas as pl
from jax.experimental.pallas import tpu as pltpu
from jax.experimental.pallas import tpu_sc as plsc

EMB = 64
NEG = 3
LANES = 16
NC = 2
NS = 16
NW = NC * NS
C = 128


def _sc_scores(targets, contexts, negsamples, in_emb, out_emb):
    B = targets.shape[0]
    per_w = B // NW
    n_rounds = per_w // C
    mesh = plsc.VectorSubcoreMesh(
        core_axis_name="c", subcore_axis_name="s", num_cores=NC, num_subcores=NS
    )

    @functools.partial(
        pl.kernel,
        out_type=(
            jax.ShapeDtypeStruct((B,), jnp.float32),
            jax.ShapeDtypeStruct((B,), jnp.float32),
        ),
        mesh=mesh,
        compiler_params=pltpu.CompilerParams(
            needs_layout_passes=False, use_tc_tiling_on_sc=False
        ),
        scratch_types=[
            pltpu.VMEM((C,), jnp.int32),
            pltpu.VMEM((C,), jnp.int32),
            pltpu.VMEM((NEG * C,), jnp.int32),
            pltpu.VMEM((C, EMB), jnp.float32),
            pltpu.VMEM((C, EMB), jnp.float32),
            pltpu.VMEM((NEG * C, EMB), jnp.float32),
            pltpu.VMEM((C * LANES,), jnp.float32),
            pltpu.VMEM((C * LANES,), jnp.float32),
            pltpu.VMEM((C,), jnp.float32),
            pltpu.VMEM((C,), jnp.float32),
            pltpu.SemaphoreType.DMA,
        ],
    )
    def scores(tg_hbm, cx_hbm, ng_hbm, ie_hbm, oe_hbm, pos_hbm, neg_hbm,
               idx_t, idx_c, idx_n, t_buf, c_buf, n_buf, pv_buf, nv_buf,
               pos_buf, neg_buf, sem):
        wid = lax.axis_index("s") * NC + lax.axis_index("c")
        wbase = wid * per_w

        def round_body(g, _):
            base = wbase + g * C
            pltpu.sync_copy(tg_hbm.at[pl.ds(base, C)], idx_t)
            pltpu.sync_copy(cx_hbm.at[pl.ds(base, C)], idx_c)
            pltpu.sync_copy(ng_hbm.at[pl.ds(NEG * base, NEG * C)], idx_n)
            cp_t = pltpu.async_copy(ie_hbm.at[idx_t], t_buf, sem)
            cp_c = pltpu.async_copy(oe_hbm.at[idx_c], c_buf, sem)
            cp_n = pltpu.async_copy(oe_hbm.at[idx_n], n_buf, sem)
            cp_t.wait()
            cp_c.wait()
            cp_n.wait()

            def item(i, _):
                t0 = t_buf[i, pl.ds(0, LANES)]
                t1 = t_buf[i, pl.ds(LANES, LANES)]
                t2 = t_buf[i, pl.ds(2 * LANES, LANES)]
                t3 = t_buf[i, pl.ds(3 * LANES, LANES)]
                pv = (t0 * c_buf[i, pl.ds(0, LANES)]
                      + t1 * c_buf[i, pl.ds(LANES, LANES)]
                      + t2 * c_buf[i, pl.ds(2 * LANES, LANES)]
                      + t3 * c_buf[i, pl.ds(3 * LANES, LANES)])
                pv_buf[pl.ds(i * LANES, LANES)] = pv
                j = i * NEG
                nv = (t0 * n_buf[j, pl.ds(0, LANES)]
                      + t1 * n_buf[j, pl.ds(LANES, LANES)]
                      + t2 * n_buf[j, pl.ds(2 * LANES, LANES)]
                      + t3 * n_buf[j, pl.ds(3 * LANES, LANES)])
                nv += (t0 * n_buf[j + 1, pl.ds(0, LANES)]
                       + t1 * n_buf[j + 1, pl.ds(LANES, LANES)]
                       + t2 * n_buf[j + 1, pl.ds(2 * LANES, LANES)]
                       + t3 * n_buf[j + 1, pl.ds(3 * LANES, LANES)])
                nv += (t0 * n_buf[j + 2, pl.ds(0, LANES)]
                       + t1 * n_buf[j + 2, pl.ds(LANES, LANES)]
                       + t2 * n_buf[j + 2, pl.ds(2 * LANES, LANES)]
                       + t3 * n_buf[j + 2, pl.ds(3 * LANES, LANES)])
                nv_buf[pl.ds(i * LANES, LANES)] = nv
                return 0

            lax.fori_loop(0, C, item, 0)

            iota16 = lax.iota(jnp.int32, LANES)

            def group(jb, _):
                rows = (jb * LANES + iota16) * LANES
                accp = plsc.load_gather(pv_buf, [rows])
                accn = plsc.load_gather(nv_buf, [rows])
                for l in range(1, LANES):
                    accp += plsc.load_gather(pv_buf, [rows + l])
                    accn += plsc.load_gather(nv_buf, [rows + l])
                pos_buf[pl.ds(jb * LANES, LANES)] = accp
                neg_buf[pl.ds(jb * LANES, LANES)] = -accn
                return 0

            lax.fori_loop(0, C // LANES, group, 0)
            pltpu.sync_copy(pos_buf, pos_hbm.at[pl.ds(base, C)])
            pltpu.sync_copy(neg_buf, neg_hbm.at[pl.ds(base, C)])
            return 0

        lax.fori_loop(0, n_rounds, round_body, 0)

    return scores(targets, contexts, negsamples, in_emb, out_emb)


def _tc_loss(pos, neg):
    B = pos.shape[0]
    p2 = pos.reshape(B // 128, 128)
    n2 = neg.reshape(B // 128, 128)

    def body(p_ref, n_ref, o_ref):
        x = jax.nn.log_sigmoid(p_ref[...]) + jax.nn.log_sigmoid(n_ref[...])
        o_ref[0, 0] = -jnp.sum(x) / B

    out = pl.pallas_call(
        body,
        out_shape=jax.ShapeDtypeStruct((1, 1), jnp.float32),
        out_specs=pl.BlockSpec(memory_space=pltpu.SMEM),
    )(p2, n2)
    return out[0, 0]


def kernel(targets, contexts, negsamples, device, in_emb, out_emb):
    del device
    pos, neg = _sc_scores(
        targets.astype(jnp.int32),
        contexts.astype(jnp.int32),
        negsamples.astype(jnp.int32),
        in_emb,
        out_emb,
    )
    return _tc_loss(pos, neg)


# speedup vs baseline: 1.5203x; 1.5203x over previous
import functools

import jax
import jax.numpy as jnp
from jax import lax
from jax.experimental import pallas as pl
from jax.experimental.pallas import tpu as pltpu
from jax.experimental.pallas import tpu_sc as plsc

EMB = 64
NEG = 3
LANES = 16
NC = 2
NS = 16
NW = NC * NS
G = 32


def _sc_scores(targets, contexts, negsamples, in_emb, out_emb):
    B = targets.shape[0]
    per_w = B // NW
    n_groups = per_w // G
    mesh = plsc.VectorSubcoreMesh(
        core_axis_name="c", subcore_axis_name="s", num_cores=NC, num_subcores=NS
    )

    @functools.partial(
        pl.kernel,
        out_type=(
            jax.ShapeDtypeStruct((B,), jnp.float32),
            jax.ShapeDtypeStruct((B,), jnp.float32),
        ),
        mesh=mesh,
        compiler_params=pltpu.CompilerParams(
            needs_layout_passes=False, use_tc_tiling_on_sc=True
        ),
        scratch_types=[
            pltpu.VMEM((G,), jnp.int32),
            pltpu.VMEM((G,), jnp.int32),
            pltpu.VMEM((NEG * G,), jnp.int32),
            pltpu.VMEM((G, EMB), jnp.float32),
            pltpu.VMEM((G, EMB), jnp.float32),
            pltpu.VMEM((NEG * G, EMB), jnp.float32),
            pltpu.VMEM((G * LANES,), jnp.float32),
            pltpu.VMEM((G * LANES,), jnp.float32),
            pltpu.VMEM((G,), jnp.float32),
            pltpu.VMEM((G,), jnp.float32),
            pltpu.SemaphoreType.DMA,
        ],
    )
    def scores(tg_hbm, cx_hbm, ng_hbm, ie_hbm, oe_hbm, pos_hbm, neg_hbm,
               idx_t, idx_c, idx_n, t_buf, c_buf, n_buf, pv_buf, nv_buf,
               pos_buf, neg_buf, sem):
        wid = lax.axis_index("s") * NC + lax.axis_index("c")
        wbase = wid * per_w

        def group_body(g, _):
            base = wbase + g * G
            pltpu.sync_copy(tg_hbm.at[pl.ds(base, G)], idx_t)
            pltpu.sync_copy(cx_hbm.at[pl.ds(base, G)], idx_c)
            pltpu.sync_copy(ng_hbm.at[pl.ds(NEG * base, NEG * G)], idx_n)

            copies = []
            for gi in range(G // LANES):
                tv = idx_t[pl.ds(gi * LANES, LANES)]
                cv = idx_c[pl.ds(gi * LANES, LANES)]
                nvs = [idx_n[pl.ds(gi * NEG * LANES + k * LANES, LANES)]
                       for k in range(NEG)]
                for ii in range(LANES):
                    i = gi * LANES + ii
                    copies.append(pltpu.async_copy(
                        ie_hbm.at[tv[ii], :],
                        t_buf.at[i, :], sem))
                    copies.append(pltpu.async_copy(
                        oe_hbm.at[cv[ii], :],
                        c_buf.at[i, :], sem))
                    for k in range(NEG):
                        j = NEG * ii + k
                        copies.append(pltpu.async_copy(
                            oe_hbm.at[nvs[j // LANES][j % LANES], :],
                            n_buf.at[NEG * i + k, :], sem))
            for cp in copies:
                cp.wait()

            def item(i, _):
                t0 = t_buf[i, pl.ds(0, LANES)]
                t1 = t_buf[i, pl.ds(LANES, LANES)]
                t2 = t_buf[i, pl.ds(2 * LANES, LANES)]
                t3 = t_buf[i, pl.ds(3 * LANES, LANES)]
                pv = (t0 * c_buf[i, pl.ds(0, LANES)]
                      + t1 * c_buf[i, pl.ds(LANES, LANES)]
                      + t2 * c_buf[i, pl.ds(2 * LANES, LANES)]
                      + t3 * c_buf[i, pl.ds(3 * LANES, LANES)])
                pv_buf[pl.ds(i * LANES, LANES)] = pv
                j = i * NEG
                nv = (t0 * n_buf[j, pl.ds(0, LANES)]
                      + t1 * n_buf[j, pl.ds(LANES, LANES)]
                      + t2 * n_buf[j, pl.ds(2 * LANES, LANES)]
                      + t3 * n_buf[j, pl.ds(3 * LANES, LANES)])
                nv += (t0 * n_buf[j + 1, pl.ds(0, LANES)]
                       + t1 * n_buf[j + 1, pl.ds(LANES, LANES)]
                       + t2 * n_buf[j + 1, pl.ds(2 * LANES, LANES)]
                       + t3 * n_buf[j + 1, pl.ds(3 * LANES, LANES)])
                nv += (t0 * n_buf[j + 2, pl.ds(0, LANES)]
                       + t1 * n_buf[j + 2, pl.ds(LANES, LANES)]
                       + t2 * n_buf[j + 2, pl.ds(2 * LANES, LANES)]
                       + t3 * n_buf[j + 2, pl.ds(3 * LANES, LANES)])
                nv_buf[pl.ds(i * LANES, LANES)] = nv
                return 0

            lax.fori_loop(0, G, item, 0)

            iota16 = lax.iota(jnp.int32, LANES)
            for jg in range(G // LANES):
                rows = (jg * LANES + iota16) * LANES
                accp = plsc.load_gather(pv_buf, [rows])
                accn = plsc.load_gather(nv_buf, [rows])
                for l in range(1, LANES):
                    accp += plsc.load_gather(pv_buf, [rows + l])
                    accn += plsc.load_gather(nv_buf, [rows + l])
                pos_buf[pl.ds(jg * LANES, LANES)] = accp
                neg_buf[pl.ds(jg * LANES, LANES)] = -accn

            pltpu.sync_copy(pos_buf, pos_hbm.at[pl.ds(base, G)])
            pltpu.sync_copy(neg_buf, neg_hbm.at[pl.ds(base, G)])
            return 0

        lax.fori_loop(0, n_groups, group_body, 0)

    return scores(targets, contexts, negsamples, in_emb, out_emb)


def _tc_loss(pos, neg):
    B = pos.shape[0]
    p2 = pos.reshape(B // 128, 128)
    n2 = neg.reshape(B // 128, 128)

    def body(p_ref, n_ref, o_ref):
        x = jax.nn.log_sigmoid(p_ref[...]) + jax.nn.log_sigmoid(n_ref[...])
        o_ref[0, 0] = -jnp.sum(x) / B

    out = pl.pallas_call(
        body,
        out_shape=jax.ShapeDtypeStruct((1, 1), jnp.float32),
        out_specs=pl.BlockSpec(memory_space=pltpu.SMEM),
    )(p2, n2)
    return out[0, 0]


def kernel(targets, contexts, negsamples, device, in_emb, out_emb):
    del device
    pos, neg = _sc_scores(
        targets.astype(jnp.int32),
        contexts.astype(jnp.int32),
        negsamples.astype(jnp.int32),
        in_emb,
        out_emb,
    )
    return _tc_loss(pos, neg)


# speedup vs baseline: 1.5932x; 1.0479x over previous
import functools

import jax
import jax.numpy as jnp
from jax import lax
from jax.experimental import pallas as pl
from jax.experimental.pallas import tpu as pltpu
from jax.experimental.pallas import tpu_sc as plsc

EMB = 64
NEG = 3
LANES = 16
NC = 2
NS = 16
NW = NC * NS
G = 64


def _sc_scores(targets, contexts, negsamples, in_emb, out_emb):
    B = targets.shape[0]
    per_w = B // NW
    n_groups = per_w // G
    mesh = plsc.VectorSubcoreMesh(
        core_axis_name="c", subcore_axis_name="s", num_cores=NC, num_subcores=NS
    )

    @functools.partial(
        pl.kernel,
        out_type=(
            jax.ShapeDtypeStruct((B,), jnp.float32),
            jax.ShapeDtypeStruct((B,), jnp.float32),
        ),
        mesh=mesh,
        compiler_params=pltpu.CompilerParams(
            needs_layout_passes=False, use_tc_tiling_on_sc=True
        ),
        scratch_types=[
            pltpu.VMEM((per_w,), jnp.int32),
            pltpu.VMEM((per_w,), jnp.int32),
            pltpu.VMEM((NEG * per_w,), jnp.int32),
            pltpu.VMEM((G, EMB), jnp.float32),
            pltpu.VMEM((G, EMB), jnp.float32),
            pltpu.VMEM((G, EMB), jnp.float32),
            pltpu.VMEM((G, EMB), jnp.float32),
            pltpu.VMEM((NEG * G, EMB), jnp.float32),
            pltpu.VMEM((NEG * G, EMB), jnp.float32),
            pltpu.VMEM((G * LANES,), jnp.float32),
            pltpu.VMEM((G * LANES,), jnp.float32),
            pltpu.VMEM((G,), jnp.float32),
            pltpu.VMEM((G,), jnp.float32),
            pltpu.SemaphoreType.DMA,
            pltpu.SemaphoreType.DMA,
            pltpu.SemaphoreType.DMA,
        ],
    )
    def scores(tg_hbm, cx_hbm, ng_hbm, ie_hbm, oe_hbm, pos_hbm, neg_hbm,
               idx_t, idx_c, idx_n, t0_buf, t1_buf, c0_buf, c1_buf,
               n0_buf, n1_buf, pv_buf, nv_buf, pos_buf, neg_buf,
               sem0, sem1, sem_io):
        wid = lax.axis_index("s") * NC + lax.axis_index("c")
        wbase = wid * per_w

        t_bufs = (t0_buf, t1_buf)
        c_bufs = (c0_buf, c1_buf)
        n_bufs = (n0_buf, n1_buf)
        sems = (sem0, sem1)

        cp1 = pltpu.async_copy(tg_hbm.at[pl.ds(wbase, per_w)], idx_t, sem_io)
        cp2 = pltpu.async_copy(cx_hbm.at[pl.ds(wbase, per_w)], idx_c, sem_io)
        cp3 = pltpu.async_copy(
            ng_hbm.at[pl.ds(NEG * wbase, NEG * per_w)], idx_n, sem_io)
        cp1.wait()
        cp2.wait()
        cp3.wait()

        def fire(g, b):
            tb, cb, nb, sem = t_bufs[b], c_bufs[b], n_bufs[b], sems[b]

            def sub(gi, _):
                ibase = gi * LANES
                tv = idx_t[pl.ds(g * G + ibase, LANES)]
                cv = idx_c[pl.ds(g * G + ibase, LANES)]
                nvs = [idx_n[pl.ds(NEG * (g * G + ibase) + k * LANES, LANES)]
                       for k in range(NEG)]
                for ii in range(LANES):
                    i = ibase + ii
                    pltpu.async_copy(ie_hbm.at[tv[ii], :], tb.at[i, :], sem)
                    pltpu.async_copy(oe_hbm.at[cv[ii], :], cb.at[i, :], sem)
                    for k in range(NEG):
                        j = NEG * ii + k
                        pltpu.async_copy(
                            oe_hbm.at[nvs[j // LANES][j % LANES], :],
                            nb.at[NEG * i + k, :], sem)
                return 0

            lax.fori_loop(0, G // LANES, sub, 0)

        def drain(b):
            tb, cb, nb, sem = t_bufs[b], c_bufs[b], n_bufs[b], sems[b]
            pltpu.make_async_copy(ie_hbm.at[pl.ds(0, G), :], tb, sem).wait()
            pltpu.make_async_copy(oe_hbm.at[pl.ds(0, G), :], cb, sem).wait()
            pltpu.make_async_copy(
                oe_hbm.at[pl.ds(0, NEG * G), :], nb, sem).wait()

        def compute(g, b):
            tb, cb, nb = t_bufs[b], c_bufs[b], n_bufs[b]

            def item(i, _):
                t0 = tb[i, pl.ds(0, LANES)]
                t1 = tb[i, pl.ds(LANES, LANES)]
                t2 = tb[i, pl.ds(2 * LANES, LANES)]
                t3 = tb[i, pl.ds(3 * LANES, LANES)]
                pv = (t0 * cb[i, pl.ds(0, LANES)]
                      + t1 * cb[i, pl.ds(LANES, LANES)]
                      + t2 * cb[i, pl.ds(2 * LANES, LANES)]
                      + t3 * cb[i, pl.ds(3 * LANES, LANES)])
                pv_buf[pl.ds(i * LANES, LANES)] = pv
                j = i * NEG
                nv = (t0 * nb[j, pl.ds(0, LANES)]
                      + t1 * nb[j, pl.ds(LANES, LANES)]
                      + t2 * nb[j, pl.ds(2 * LANES, LANES)]
                      + t3 * nb[j, pl.ds(3 * LANES, LANES)])
                nv += (t0 * nb[j + 1, pl.ds(0, LANES)]
                       + t1 * nb[j + 1, pl.ds(LANES, LANES)]
                       + t2 * nb[j + 1, pl.ds(2 * LANES, LANES)]
                       + t3 * nb[j + 1, pl.ds(3 * LANES, LANES)])
                nv += (t0 * nb[j + 2, pl.ds(0, LANES)]
                       + t1 * nb[j + 2, pl.ds(LANES, LANES)]
                       + t2 * nb[j + 2, pl.ds(2 * LANES, LANES)]
                       + t3 * nb[j + 2, pl.ds(3 * LANES, LANES)])
                nv_buf[pl.ds(i * LANES, LANES)] = nv
                return 0

            lax.fori_loop(0, G, item, 0)

            iota16 = lax.iota(jnp.int32, LANES)

            def red(jg, _):
                rows = (jg * LANES + iota16) * LANES
                accp = plsc.load_gather(pv_buf, [rows])
                accn = plsc.load_gather(nv_buf, [rows])
                for l in range(1, LANES):
                    accp += plsc.load_gather(pv_buf, [rows + l])
                    accn += plsc.load_gather(nv_buf, [rows + l])
                pos_buf[pl.ds(jg * LANES, LANES)] = accp
                neg_buf[pl.ds(jg * LANES, LANES)] = -accn
                return 0

            lax.fori_loop(0, G // LANES, red, 0)
            base = wbase + g * G
            pltpu.sync_copy(pos_buf, pos_hbm.at[pl.ds(base, G)])
            pltpu.sync_copy(neg_buf, neg_hbm.at[pl.ds(base, G)])

        fire(0, 0)

        def pair(gg, _):
            g0 = 2 * gg
            fire(g0 + 1, 1)
            drain(0)
            compute(g0, 0)

            @pl.when(gg < n_groups // 2 - 1)
            def _():
                fire(g0 + 2, 0)

            drain(1)
            compute(g0 + 1, 1)
            return 0

        lax.fori_loop(0, n_groups // 2, pair, 0)

    return scores(targets, contexts, negsamples, in_emb, out_emb)


def _tc_loss(pos, neg):
    B = pos.shape[0]
    p2 = pos.reshape(B // 128, 128)
    n2 = neg.reshape(B // 128, 128)

    def body(p_ref, n_ref, o_ref):
        x = jax.nn.log_sigmoid(p_ref[...]) + jax.nn.log_sigmoid(n_ref[...])
        o_ref[0, 0] = -jnp.sum(x) / B

    out = pl.pallas_call(
        body,
        out_shape=jax.ShapeDtypeStruct((1, 1), jnp.float32),
        out_specs=pl.BlockSpec(memory_space=pltpu.SMEM),
    )(p2, n2)
    return out[0, 0]


def kernel(targets, contexts, negsamples, device, in_emb, out_emb):
    del device
    pos, neg = _sc_scores(
        targets.astype(jnp.int32),
        contexts.astype(jnp.int32),
        negsamples.astype(jnp.int32),
        in_emb,
        out_emb,
    )
    return _tc_loss(pos, neg)
